# Initial kernel scaffold; baseline (speedup 1.0000x reference)
import jax
import jax.numpy as jnp
from jax.experimental import pallas as pl


def kernel(x, time, token_table, time_table):
    raise NotImplementedError("write your pallas kernel here")


# speedup vs baseline: 1.0161x; 1.0161x over previous
import functools
import math

import jax
import jax.numpy as jnp
import numpy as np
from jax import lax
from jax.experimental import pallas as pl
from jax.experimental.pallas import tpu as pltpu
from jax.experimental.pallas import tpu_sc as plsc

D_MODEL = 64
SEQ = 200
SCALE = 8.0
NC = 2
NS = 16
NW = NC * NS
CH = 128
LANES = 16


def _pe_scaled_dup():
    position = np.arange(0, SEQ, dtype=np.float32)[:, None]
    div = np.exp(
        np.arange(0, D_MODEL, 2, dtype=np.float32) * -(math.log(10000.0) / D_MODEL)
    )
    pe = np.zeros((SEQ, D_MODEL), dtype=np.float32)
    pe[:, 0::2] = np.sin(position * div)
    pe[:, 1::2] = np.cos(position * div)
    pe = pe * np.float32(SCALE)
    return jnp.asarray(np.concatenate([pe, pe], axis=0))


def _make_sc_embed(n_rows):
    rows_per_w = n_rows // NW
    n_chunks = rows_per_w // CH
    mesh = plsc.VectorSubcoreMesh(core_axis_name="c", subcore_axis_name="s")

    @functools.partial(
        pl.kernel,
        out_type=jax.ShapeDtypeStruct((n_rows, D_MODEL), jnp.float32),
        mesh=mesh,
        compiler_params=pltpu.CompilerParams(use_tc_tiling_on_sc=False),
        scratch_types=[
            pltpu.VMEM((CH,), jnp.int32),
            pltpu.VMEM((CH,), jnp.int32),
            pltpu.VMEM((CH, D_MODEL), jnp.float32),
            pltpu.VMEM((CH, D_MODEL), jnp.float32),
            pltpu.VMEM((2 * SEQ, D_MODEL), jnp.float32),
            pltpu.SemaphoreType.DMA,
            pltpu.SemaphoreType.DMA,
        ],
    )
    def sc_embed(xf, tf, tok_tab, time_tab8, pe8, out,
                 idx_v, tidx_v, tok_v, time_v, pe_v, sem_t, sem_m):
        wid = lax.axis_index("s") * NC + lax.axis_index("c")
        base0 = wid * rows_per_w
        pltpu.sync_copy(pe8, pe_v)

        def chunk_body(c, carry):
            base = base0 + c * CH
            s_off = lax.rem(base, SEQ)
            pltpu.sync_copy(xf.at[pl.ds(base, CH)], idx_v)
            pltpu.sync_copy(tf.at[pl.ds(base, CH)], tidx_v)
            ct = pltpu.async_copy(tok_tab.at[idx_v], tok_v, sem_t)
            cm = pltpu.async_copy(time_tab8.at[tidx_v], time_v, sem_m)
            ct.wait()
            cm.wait()

            def row_body(r, rcarry):
                pr = s_off + r
                for j in range(D_MODEL // LANES):
                    sl = pl.ds(j * LANES, LANES)
                    tok_v[r, sl] = (
                        tok_v[r, sl] * SCALE + time_v[r, sl] + pe_v[pr, sl]
                    )
                return rcarry

            lax.fori_loop(0, CH, row_body, 0)
            pltpu.sync_copy(tok_v, out.at[pl.ds(base, CH)])
            return carry

        lax.fori_loop(0, n_chunks, chunk_body, 0)

    return sc_embed


_sc_embed_204800 = _make_sc_embed(1024 * SEQ)


def kernel(x, time, token_table, time_table):
    b, s = x.shape
    xf = x.reshape(-1)
    tf = time.reshape(-1)
    tt8 = (time_table * jnp.float32(SCALE)).astype(jnp.float32)
    pe8 = _pe_scaled_dup()
    out = _sc_embed_204800(xf, tf, token_table, tt8, pe8)
    return out.reshape(b, s, D_MODEL)
